# baseline (speedup 1.0000x reference)
import dataclasses

import jax
import jax.numpy as jnp
from jax import lax
from jax.experimental import pallas as pl
from jax.experimental.pallas import tpu as pltpu
from jax.experimental.pallas import tpu_sc as plsc

NB = 16
MV = 4096
N = NB * 2048
OUTN = N + NB * MV
SENT = 0x00FFFFFF

CNT_ADJ = 0
RANK_SUB = 1


def _spread3(v):
    v = (v | (v << 16)) & 0x030000FF
    v = (v | (v << 8)) & 0x0300F00F
    v = (v | (v << 4)) & 0x030C30C3
    v = (v | (v << 2)) & 0x09249249
    return v


def _tc_body(cols_ref, f2w_ref, w2f_ref, mask_ref, codes_ref, bs_ref):
    b = cols_ref[0]
    z = cols_ref[1]
    y = cols_ref[2]
    x = cols_ref[3]
    bs = [jnp.int32(0)]
    for k in range(1, NB + 1):
        bs.append(jnp.sum((b < k).astype(jnp.int32)))
    adj = jnp.zeros((256, 128), jnp.int32)
    for k in range(NB):
        adj = jnp.where(b == k, k * MV - bs[k], adj)
    row = lax.broadcasted_iota(jnp.int32, (256, 128), 0)
    col = lax.broadcasted_iota(jnp.int32, (256, 128), 1)
    w2f_ref[...] = row * 128 + col + adj
    lrow = lax.broadcasted_iota(jnp.int32, (32, 128), 0)
    lcol = lax.broadcasted_iota(jnp.int32, (32, 128), 1)
    off = lrow * 128 + lcol
    for k in range(NB):
        n = bs[k + 1] - bs[k]
        st = bs[k]
        period = 2 * n - 2
        t = jnp.maximum(off - n, 0)
        pf = period.astype(jnp.float32)
        q = jnp.floor(t.astype(jnp.float32) / pf).astype(jnp.int32)
        r = t - q * period
        q = q + jnp.where(r >= period, 1, 0) - jnp.where(r < 0, 1, 0)
        m = t - q * period
        mirror = jnp.where(m < n - 1, n - 2 - m, m - n + 2)
        f2w_ref[32 * k:32 * (k + 1), :] = st + jnp.where(off < n, off, mirror)
        mask_ref[32 * k:32 * (k + 1), :] = off >= n
    sz = _spread3(z)
    sy = _spread3(y)
    sx = _spread3(x)
    codes_ref[0] = sz | (sy << 1) | (sx << 2)
    codes_ref[1] = sz | (sx << 1) | (sy << 2)
    lane = lax.broadcasted_iota(jnp.int32, (8, 128), 1)
    acc = jnp.zeros((8, 128), jnp.int32)
    for k in range(NB + 1):
        acc = jnp.where(lane == k, bs[k], acc)
    bs_ref[...] = acc


def _sc_body(codes_hbm, bs_hbm, out_hbm,
             keys_a, vals_a, keys_b, vals_b, hist1, hist2, oidx, bs_v):
    c = lax.axis_index("c")
    s = lax.axis_index("s")
    pltpu.sync_copy(bs_hbm.at[0, pl.ds(0, 32)], bs_v)
    iota = lax.iota(jnp.int32, 16)
    v0 = bs_v[0:16]
    v1 = bs_v[16:32]
    zero = jnp.zeros((16,), jnp.int32)
    s_b = jnp.sum(jnp.where(iota == s, v0, zero))
    e_b = (jnp.sum(jnp.where(iota == s + 1, v0, zero))
           + jnp.sum(jnp.where(iota == s - 15, v1, zero)))
    n_b = e_b - s_b
    start = pl.multiple_of(jnp.minimum(s_b, N - MV), 8)
    pltpu.sync_copy(codes_hbm.at[pl.ds(pl.multiple_of(c * N + start, 8), MV)],
                    keys_a)

    @pl.loop(0, MV, step=16)
    def _(i):
        hist1[pl.ds(i, 16)] = zero
        hist2[pl.ds(i, 16)] = zero

    @pl.loop(0, MV, step=16)
    def _(i):
        k = keys_a[pl.ds(i, 16)]
        g = start + i + iota
        valid = jnp.logical_and(g >= s_b, g < e_b)
        k = jnp.where(valid, k, SENT)
        keys_a[pl.ds(i, 16)] = k
        vals_a[pl.ds(i, 16)] = g
        d1 = k & 0xFFF
        cnt1, last1 = plsc.scan_count(d1)
        plsc.addupdate_scatter(hist1, [d1], cnt1 + CNT_ADJ, mask=last1)
        d2 = k >> 12
        cnt2, last2 = plsc.scan_count(d2)
        plsc.addupdate_scatter(hist2, [d2], cnt2 + CNT_ADJ, mask=last2)

    def scan_body(j, carry):
        c1, c2 = carry
        h1 = hist1[pl.ds(j * 16, 16)]
        cs1 = plsc.cumsum(h1)
        hist1[pl.ds(j * 16, 16)] = cs1 - h1 + c1
        h2 = hist2[pl.ds(j * 16, 16)]
        cs2 = plsc.cumsum(h2)
        hist2[pl.ds(j * 16, 16)] = cs2 - h2 + c2
        return (c1 + jnp.sum(h1), c2 + jnp.sum(h2))

    lax.fori_loop(0, MV // 16, scan_body, (jnp.int32(0), jnp.int32(0)))

    @pl.loop(0, MV, step=16)
    def _(i):
        k = keys_a[pl.ds(i, 16)]
        v = vals_a[pl.ds(i, 16)]
        d1 = k & 0xFFF
        cnt, last = plsc.scan_count(d1)
        base = plsc.load_gather(hist1, [d1])
        pos = base + cnt - RANK_SUB
        plsc.store_scatter(keys_b, [pos], k)
        plsc.store_scatter(vals_b, [pos], v)
        plsc.addupdate_scatter(hist1, [d1], cnt + CNT_ADJ, mask=last)

    @pl.loop(0, MV, step=16)
    def _(i):
        k = keys_b[pl.ds(i, 16)]
        v = vals_b[pl.ds(i, 16)]
        d2 = k >> 12
        cnt, last = plsc.scan_count(d2)
        base = plsc.load_gather(hist2, [d2])
        pos = base + cnt - RANK_SUB
        plsc.store_scatter(vals_a, [pos], v)
        plsc.addupdate_scatter(hist2, [d2], cnt + CNT_ADJ, mask=last)

    @pl.loop(0, MV, step=16)
    def _(i):
        li = i + iota
        oidx[pl.ds(i, 16)] = (c * OUTN
                              + jnp.where(li < n_b, s_b + li, N + s * MV + li))

    pltpu.sync_copy(vals_a, out_hbm.at[oidx])


def _run_sc_sort(codes, bsgrid):
    mesh = plsc.VectorSubcoreMesh(core_axis_name="c", subcore_axis_name="s")
    cp = pltpu.CompilerParams()
    if "needs_layout_passes" in pltpu.CompilerParams.__dataclass_fields__:
        cp = dataclasses.replace(cp, needs_layout_passes=False)
    f = pl.kernel(
        _sc_body,
        out_type=jax.ShapeDtypeStruct((2 * OUTN,), jnp.int32),
        mesh=mesh,
        scratch_types=[
            pltpu.VMEM((MV,), jnp.int32),
            pltpu.VMEM((MV,), jnp.int32),
            pltpu.VMEM((MV,), jnp.int32),
            pltpu.VMEM((MV,), jnp.int32),
            pltpu.VMEM((MV,), jnp.int32),
            pltpu.VMEM((MV,), jnp.int32),
            pltpu.VMEM((MV,), jnp.int32),
            pltpu.VMEM((32,), jnp.int32),
        ],
        compiler_params=cp,
    )
    return f(codes, bsgrid)


def kernel(coords, batch_size, max_voxels, sparse_shape):
    del batch_size, max_voxels, sparse_shape
    cols = coords.astype(jnp.int32).T.reshape(4, 256, 128)
    f2w, w2f, mask2d, codes, bsgrid = pl.pallas_call(
        _tc_body,
        out_shape=[
            jax.ShapeDtypeStruct((512, 128), jnp.int32),
            jax.ShapeDtypeStruct((256, 128), jnp.int32),
            jax.ShapeDtypeStruct((512, 128), jnp.bool_),
            jax.ShapeDtypeStruct((2, 256, 128), jnp.int32),
            jax.ShapeDtypeStruct((8, 128), jnp.int32),
        ],
    )(cols)
    out = _run_sc_sort(codes.reshape(2 * N), bsgrid)
    return (f2w.reshape(-1), w2f.reshape(-1), mask2d.reshape(-1),
            out[:N], out[OUTN:OUTN + N])
